# baseline (speedup 1.0000x reference)
import functools

import jax
import jax.numpy as jnp
from jax import lax
from jax.experimental import pallas as pl
from jax.experimental.pallas import tpu as pltpu
from jax.experimental.pallas import tpu_sc as plsc

NC = 2
NS = 16

EMBED_DIM = 128
SEQ = 200
HALF = 100
NBUF = 4
K_INIT = 3
K_GATH = 2


def _embed_kernel(n_rows_per_w, x_hbm, tok_hbm, pos_hbm, out_hbm,
                  idx_v, pos_sh, buf, gsem, ssem, isem):
    wid = lax.axis_index("s") * NC + lax.axis_index("c")
    row0 = wid * n_rows_per_w

    pltpu.sync_copy(x_hbm.at[pl.ds(row0 * 2, n_rows_per_w * 2)], idx_v)

    @pl.when(lax.axis_index("s") == 0)
    def _():
        pltpu.sync_copy(pos_hbm, pos_sh)

    plsc.subcore_barrier()

    def fire_init(b):
        pltpu.async_copy(pos_sh, buf.at[b], isem[b])

    def wait_init(b):
        pltpu.make_async_copy(pos_sh, buf.at[b], isem[b]).wait()

    def fire_gathers(r, b):
        for h in range(2):
            pltpu.async_copy(tok_hbm.at[idx_v.at[2 * r + h]],
                             buf.at[b, pl.ds(h * HALF, HALF)],
                             gsem[b], add=True)

    def wait_gathers(r, b):
        for h in range(2):
            pltpu.make_async_copy(tok_hbm.at[idx_v.at[2 * r + h]],
                                  buf.at[b, pl.ds(h * HALF, HALF)],
                                  gsem[b]).wait()

    def fire_store(r, b):
        pltpu.async_copy(buf.at[b],
                         out_hbm.at[pl.ds((row0 + r) * SEQ, SEQ)],
                         ssem[b])

    def wait_store(r, b):
        pltpu.make_async_copy(buf.at[b],
                              out_hbm.at[pl.ds((row0 + r) * SEQ, SEQ)],
                              ssem[b]).wait()

    for r in range(K_INIT):
        fire_init(r % NBUF)
    for r in range(K_GATH):
        wait_init(r % NBUF)
        fire_gathers(r, r % NBUF)

    def step(g, carry):
        for b0 in range(NBUF):
            r = g * NBUF + b0
            wait_gathers(r, b0)
            fire_store(r, b0)

            ri = r + K_INIT
            bi = (b0 + K_INIT) % NBUF

            @pl.when(ri < n_rows_per_w)
            def _():
                @pl.when(ri >= NBUF)
                def _():
                    wait_store(ri - NBUF, bi)
                fire_init(bi)

            rg = r + K_GATH
            bg = (b0 + K_GATH) % NBUF

            @pl.when(rg < n_rows_per_w)
            def _():
                wait_init(bg)
                fire_gathers(rg, bg)
        return carry

    lax.fori_loop(0, n_rows_per_w // NBUF, step, 0)

    for b in range(NBUF):
        r = n_rows_per_w - NBUF + b
        wait_store(r, b)


def kernel(x, token_table, pos_table):
    B, S = x.shape
    D = token_table.shape[1]
    n_lookups = B * S
    n_w = NC * NS
    n_rows_per_w = B // n_w

    x_rows = x.reshape(n_lookups // HALF, HALF).astype(jnp.int32)

    mesh = plsc.VectorSubcoreMesh(
        core_axis_name="c", subcore_axis_name="s",
        num_cores=NC, num_subcores=NS)

    out_flat = pl.kernel(
        functools.partial(_embed_kernel, n_rows_per_w),
        out_type=jax.ShapeDtypeStruct((n_lookups, D), jnp.float32),
        mesh=mesh,
        scratch_types=[
            pltpu.VMEM((n_rows_per_w * 2, HALF), jnp.int32),
            pltpu.VMEM_SHARED((S, D), jnp.float32),
            pltpu.VMEM((NBUF, SEQ, D), jnp.float32),
            [pltpu.SemaphoreType.DMA] * NBUF,
            [pltpu.SemaphoreType.DMA] * NBUF,
            [pltpu.SemaphoreType.DMA] * NBUF,
        ],
        compiler_params=pltpu.CompilerParams(use_tc_tiling_on_sc=False),
    )(x_rows, token_table, pos_table)

    return out_flat.reshape(B, S, D)


# speedup vs baseline: 1.0279x; 1.0279x over previous
import functools

import jax
import jax.numpy as jnp
from jax import lax
from jax.experimental import pallas as pl
from jax.experimental.pallas import tpu as pltpu
from jax.experimental.pallas import tpu_sc as plsc

NC = 2
NS = 16

EMBED_DIM = 128
CHUNK = 100
NBUF = 8
K_INIT = 6
K_GATH = 4


def _embed_kernel(n_chunks_per_w, x_hbm, tok_hbm, pos_hbm, out_hbm,
                  idx_v, pos_sh, buf, gsem, ssem, isem, xsem):
    wid = lax.axis_index("s") * NC + lax.axis_index("c")
    row0 = wid * n_chunks_per_w

    idx_cp = pltpu.async_copy(x_hbm.at[pl.ds(row0, n_chunks_per_w)],
                              idx_v, xsem)

    @pl.when(lax.axis_index("s") == 0)
    def _():
        pltpu.sync_copy(pos_hbm, pos_sh)

    plsc.subcore_barrier()
    idx_cp.wait()

    def pos_off(c):
        return lax.rem(c, 2) * CHUNK

    def fire_init(c, b):
        pltpu.async_copy(pos_sh.at[pl.ds(pos_off(c), CHUNK)],
                         buf.at[b], isem[b])

    def wait_init(c, b):
        pltpu.make_async_copy(pos_sh.at[pl.ds(pos_off(c), CHUNK)],
                              buf.at[b], isem[b]).wait()

    def fire_gather(c, b):
        pltpu.async_copy(tok_hbm.at[idx_v.at[c]], buf.at[b], gsem[b],
                         add=True)

    def wait_gather(c, b):
        pltpu.make_async_copy(tok_hbm.at[idx_v.at[c]],
                              buf.at[b], gsem[b]).wait()

    def fire_store(c, b):
        pltpu.async_copy(buf.at[b],
                         out_hbm.at[pl.ds((row0 + c) * CHUNK, CHUNK)],
                         ssem[b])

    def wait_store(c, b):
        pltpu.make_async_copy(buf.at[b],
                              out_hbm.at[pl.ds((row0 + c) * CHUNK, CHUNK)],
                              ssem[b]).wait()

    for c in range(K_INIT):
        fire_init(c, c % NBUF)
    for c in range(K_GATH):
        wait_init(c, c % NBUF)
        fire_gather(c, c % NBUF)

    def step(g, carry):
        for b0 in range(NBUF):
            c = g * NBUF + b0
            wait_gather(c, b0)
            fire_store(c, b0)

            ci = c + K_INIT
            bi = (b0 + K_INIT) % NBUF

            @pl.when(ci < n_chunks_per_w)
            def _():
                @pl.when(ci >= NBUF)
                def _():
                    wait_store(ci - NBUF, bi)
                fire_init(ci, bi)

            cg = c + K_GATH
            bg = (b0 + K_GATH) % NBUF

            @pl.when(cg < n_chunks_per_w)
            def _():
                wait_init(cg, bg)
                fire_gather(cg, bg)
        return carry

    lax.fori_loop(0, n_chunks_per_w // NBUF, step, 0)

    for b in range(NBUF):
        c = n_chunks_per_w - NBUF + b
        wait_store(c, b)


def kernel(x, token_table, pos_table):
    B, S = x.shape
    D = token_table.shape[1]
    n_lookups = B * S
    n_w = NC * NS
    n_chunks = n_lookups // CHUNK
    n_chunks_per_w = n_chunks // n_w

    x_rows = x.reshape(n_chunks, CHUNK).astype(jnp.int32)

    mesh = plsc.VectorSubcoreMesh(
        core_axis_name="c", subcore_axis_name="s",
        num_cores=NC, num_subcores=NS)

    out_flat = pl.kernel(
        functools.partial(_embed_kernel, n_chunks_per_w),
        out_type=jax.ShapeDtypeStruct((n_lookups, D), jnp.float32),
        mesh=mesh,
        scratch_types=[
            pltpu.VMEM((n_chunks_per_w, CHUNK), jnp.int32),
            pltpu.VMEM_SHARED((S, D), jnp.float32),
            pltpu.VMEM((NBUF, CHUNK, D), jnp.float32),
            [pltpu.SemaphoreType.DMA] * NBUF,
            [pltpu.SemaphoreType.DMA] * NBUF,
            [pltpu.SemaphoreType.DMA] * NBUF,
            pltpu.SemaphoreType.DMA,
        ],
        compiler_params=pltpu.CompilerParams(use_tc_tiling_on_sc=False),
    )(x_rows, token_table, pos_table)

    return out_flat.reshape(B, S, D)


# speedup vs baseline: 1.0343x; 1.0062x over previous
import functools

import jax
import jax.numpy as jnp
from jax import lax
from jax.experimental import pallas as pl
from jax.experimental.pallas import tpu as pltpu
from jax.experimental.pallas import tpu_sc as plsc

NC = 2
NS = 16

EMBED_DIM = 128
CHUNK = 100
NBUF = 8
K_INIT = 6
K_GATH = 4


def _embed_kernel(n_chunks_per_w, x_hbm, tok_hbm, pos_hbm, out_hbm,
                  idx_v, pos_sh, buf, gsem, ssem, isem, xsem):
    wid = lax.axis_index("s") * NC + lax.axis_index("c")
    row0 = wid * n_chunks_per_w

    idx_cp = pltpu.async_copy(x_hbm.at[pl.ds(row0, n_chunks_per_w)],
                              idx_v, xsem)

    @pl.when(lax.axis_index("s") == 0)
    def _():
        pltpu.sync_copy(pos_hbm, pos_sh)

    plsc.subcore_barrier()
    idx_cp.wait()

    n_w = NC * NS

    def out_base(c):
        return (c * n_w + wid) * CHUNK

    def pos_off(c):
        return lax.rem(wid, 2) * CHUNK

    def fire_init(c, b):
        pltpu.async_copy(pos_sh.at[pl.ds(pos_off(c), CHUNK)],
                         buf.at[b], isem[b])

    def wait_init(c, b):
        pltpu.make_async_copy(pos_sh.at[pl.ds(pos_off(c), CHUNK)],
                              buf.at[b], isem[b]).wait()

    def fire_gather(c, b):
        pltpu.async_copy(tok_hbm.at[idx_v.at[c]], buf.at[b], gsem[b],
                         add=True)

    def wait_gather(c, b):
        pltpu.make_async_copy(tok_hbm.at[idx_v.at[c]],
                              buf.at[b], gsem[b]).wait()

    def fire_store(c, b):
        pltpu.async_copy(buf.at[b],
                         out_hbm.at[pl.ds(out_base(c), CHUNK)],
                         ssem[b])

    def wait_store(c, b):
        pltpu.make_async_copy(buf.at[b],
                              out_hbm.at[pl.ds(out_base(c), CHUNK)],
                              ssem[b]).wait()

    for c in range(K_INIT):
        fire_init(c, c % NBUF)
    for c in range(K_GATH):
        wait_init(c, c % NBUF)
        fire_gather(c, c % NBUF)

    def step(g, carry):
        for b0 in range(NBUF):
            c = g * NBUF + b0
            wait_gather(c, b0)
            fire_store(c, b0)

            ci = c + K_INIT
            bi = (b0 + K_INIT) % NBUF

            @pl.when(ci < n_chunks_per_w)
            def _():
                @pl.when(ci >= NBUF)
                def _():
                    wait_store(ci - NBUF, bi)
                fire_init(ci, bi)

            cg = c + K_GATH
            bg = (b0 + K_GATH) % NBUF

            @pl.when(cg < n_chunks_per_w)
            def _():
                wait_init(cg, bg)
                fire_gather(cg, bg)
        return carry

    lax.fori_loop(0, n_chunks_per_w // NBUF, step, 0)

    for b in range(NBUF):
        c = n_chunks_per_w - NBUF + b
        wait_store(c, b)


def kernel(x, token_table, pos_table):
    B, S = x.shape
    D = token_table.shape[1]
    n_lookups = B * S
    n_w = NC * NS
    n_chunks = n_lookups // CHUNK
    n_chunks_per_w = n_chunks // n_w

    x_rows = (x.reshape(n_chunks // n_w, n_w, CHUNK)
              .transpose(1, 0, 2).reshape(n_chunks, CHUNK).astype(jnp.int32))

    mesh = plsc.VectorSubcoreMesh(
        core_axis_name="c", subcore_axis_name="s",
        num_cores=NC, num_subcores=NS)

    out_flat = pl.kernel(
        functools.partial(_embed_kernel, n_chunks_per_w),
        out_type=jax.ShapeDtypeStruct((n_lookups, D), jnp.float32),
        mesh=mesh,
        scratch_types=[
            pltpu.VMEM((n_chunks_per_w, CHUNK), jnp.int32),
            pltpu.VMEM_SHARED((S, D), jnp.float32),
            pltpu.VMEM((NBUF, CHUNK, D), jnp.float32),
            [pltpu.SemaphoreType.DMA] * NBUF,
            [pltpu.SemaphoreType.DMA] * NBUF,
            [pltpu.SemaphoreType.DMA] * NBUF,
            pltpu.SemaphoreType.DMA,
        ],
        compiler_params=pltpu.CompilerParams(use_tc_tiling_on_sc=False),
    )(x_rows, token_table, pos_table)

    return out_flat.reshape(B, S, D)
